# baseline (speedup 1.0000x reference)
import jax
import jax.numpy as jnp
from jax.experimental import pallas as pl
from jax.experimental.pallas import tpu as pltpu

_B = 128
_F = 16
_H = 32
_J = 8
_C = 4
_Z = 8
_G = 32

_RW1H = 0
_RW2H = 16
_RW1G = 48
_RW2G = 56
_RB = 88
_WROWS = 96


def _fused_kernel(x_ref, z_ref, w_ref, o_ref):
    G, B, F, H, J, C, Z = _G, _B, _F, _H, _J, _C, _Z

    X = x_ref[...].reshape(G * B, F)
    Zm = z_ref[...].reshape(G * B, Z)

    w1h = w_ref[_RW1H:_RW1H + F, :]
    w2h = w_ref[_RW2H:_RW2H + H, :]
    w1g = w_ref[_RW1G:_RW1G + Z, :]
    w2g = w_ref[_RW2G:_RW2G + H, :]
    b1h = w_ref[_RB + 0:_RB + 1, :]
    b2h = w_ref[_RB + 1:_RB + 2, :]
    b1g = w_ref[_RB + 2:_RB + 3, :]
    b2g = w_ref[_RB + 3:_RB + 4, :]

    A = jnp.maximum(jnp.dot(X, w1h, preferred_element_type=jnp.float32) + b1h, 0.0)
    Hf = jnp.dot(A, w2h, preferred_element_type=jnp.float32) + b2h
    Ag = jnp.maximum(jnp.dot(Zm, w1g, preferred_element_type=jnp.float32) + b1g, 0.0)
    Gr = jnp.tanh(jnp.dot(Ag, w2g, preferred_element_type=jnp.float32) + b2g)

    for e in range(G):
        h_e = Hf[e * B:(e + 1) * B, 0:J]
        g_e = Gr[e * B:(e + 1) * B, :]
        gbig = jnp.concatenate(
            [g_e[:, c * J:(c + 1) * J].T for c in range(C)], axis=1)
        s = jnp.dot(h_e, gbig, preferred_element_type=jnp.float32)
        m = jnp.max(s, axis=0, keepdims=True)
        e_ = s - m
        lse = jnp.log(jnp.sum(jnp.exp(e_), axis=0, keepdims=True))
        o_ref[e, :, :] = (e_ - lse).astype(jnp.bfloat16)


def kernel(x, z, wslab):
    N, B = x.shape[0], x.shape[1]
    C, J, H, F, Z = _C, _J, _H, _F, _Z

    wp = jnp.zeros((_WROWS, H), jnp.float32)
    wp = wp.at[_RW1H:_RW1H + F, :].set(wslab[0:16, :])
    wp = wp.at[_RW2H:_RW2H + H, :].set(wslab[16:48, :])
    wp = wp.at[_RW1G:_RW1G + Z, :].set(wslab[48:80, 0:Z].T)
    wp = wp.at[_RW2G:_RW2G + H, :].set(wslab[80:112, :].T)
    wp = wp.at[_RB + 0, :].set(wslab[112, :])
    wp = wp.at[_RB + 1, :].set(wslab[120, :])
    wp = wp.at[_RB + 2, :].set(wslab[128:160, 0])
    wp = wp.at[_RB + 3, :].set(wslab[160:192, 0])

    out = pl.pallas_call(
        _fused_kernel,
        grid=(N // _G,),
        in_specs=[
            pl.BlockSpec((_G, B, F), lambda n: (n, 0, 0)),
            pl.BlockSpec((_G, B, Z), lambda n: (n, 0, 0)),
            pl.BlockSpec((_WROWS, H), lambda n: (0, 0)),
        ],
        out_specs=pl.BlockSpec((_G, B, C * B), lambda n: (n, 0, 0)),
        out_shape=jax.ShapeDtypeStruct((N, B, C * B), jnp.bfloat16),
        compiler_params=pltpu.CompilerParams(dimension_semantics=("parallel",)),
    )(x, z, wp)

    one = (wslab[0, 0] * 0.0 + 1.0).astype(jnp.bfloat16)
    f = jnp.transpose(out.reshape(N, B, C, B), (0, 3, 1, 2)) * one
    return f.reshape(N, B * B, C).astype(jnp.float32)


# speedup vs baseline: 1.1827x; 1.1827x over previous
import jax
import jax.numpy as jnp
from jax.experimental import pallas as pl
from jax.experimental.pallas import tpu as pltpu

_B = 128
_F = 16
_H = 32
_J = 8
_C = 4
_Z = 8
_G = 64

_RW1H = 0
_RW2H = 16
_RW1G = 48
_RW2G = 56
_RB = 88
_WROWS = 96


def _fused_kernel(x_ref, z_ref, w_ref, o_ref):
    G, B, F, H, J, C, Z = _G, _B, _F, _H, _J, _C, _Z

    X = x_ref[...].reshape(G * B, F)
    Zm = z_ref[...].reshape(G * B, Z)

    w1h = w_ref[_RW1H:_RW1H + F, :]
    w2h = w_ref[_RW2H:_RW2H + H, :]
    w1g = w_ref[_RW1G:_RW1G + Z, :]
    w2g = w_ref[_RW2G:_RW2G + H, :]
    b1h = w_ref[_RB + 0:_RB + 1, :]
    b2h = w_ref[_RB + 1:_RB + 2, :]
    b1g = w_ref[_RB + 2:_RB + 3, :]
    b2g = w_ref[_RB + 3:_RB + 4, :]

    A = jnp.maximum(jnp.dot(X, w1h, preferred_element_type=jnp.float32) + b1h, 0.0)
    Hf = jnp.dot(A, w2h, preferred_element_type=jnp.float32) + b2h
    Ag = jnp.maximum(jnp.dot(Zm, w1g, preferred_element_type=jnp.float32) + b1g, 0.0)
    Gr = jnp.tanh(jnp.dot(Ag, w2g, preferred_element_type=jnp.float32) + b2g)

    for e in range(G):
        h_e = Hf[e * B:(e + 1) * B, 0:J]
        g_e = Gr[e * B:(e + 1) * B, :]
        gbig = jnp.concatenate(
            [g_e[:, c * J:(c + 1) * J].T for c in range(C)], axis=1)
        s = jnp.dot(h_e, gbig, preferred_element_type=jnp.float32)
        m = jnp.max(s, axis=0, keepdims=True)
        e_ = s - m
        lse = jnp.log(jnp.sum(jnp.exp(e_), axis=0, keepdims=True))
        o_ref[e, :, :] = (e_ - lse).astype(jnp.bfloat16)


def kernel(x, z, wslab):
    N, B = x.shape[0], x.shape[1]
    C, J, H, F, Z = _C, _J, _H, _F, _Z

    wp = jnp.zeros((_WROWS, H), jnp.float32)
    wp = wp.at[_RW1H:_RW1H + F, :].set(wslab[0:16, :])
    wp = wp.at[_RW2H:_RW2H + H, :].set(wslab[16:48, :])
    wp = wp.at[_RW1G:_RW1G + Z, :].set(wslab[48:80, 0:Z].T)
    wp = wp.at[_RW2G:_RW2G + H, :].set(wslab[80:112, :].T)
    wp = wp.at[_RB + 0, :].set(wslab[112, :])
    wp = wp.at[_RB + 1, :].set(wslab[120, :])
    wp = wp.at[_RB + 2, :].set(wslab[128:160, 0])
    wp = wp.at[_RB + 3, :].set(wslab[160:192, 0])

    out = pl.pallas_call(
        _fused_kernel,
        grid=(N // _G,),
        in_specs=[
            pl.BlockSpec((_G, B, F), lambda n: (n, 0, 0)),
            pl.BlockSpec((_G, B, Z), lambda n: (n, 0, 0)),
            pl.BlockSpec((_WROWS, H), lambda n: (0, 0)),
        ],
        out_specs=pl.BlockSpec((_G, B, C * B), lambda n: (n, 0, 0)),
        out_shape=jax.ShapeDtypeStruct((N, B, C * B), jnp.bfloat16),
        compiler_params=pltpu.CompilerParams(dimension_semantics=("parallel",)),
    )(x, z, wp)

    f = jnp.transpose(out.reshape(N, B, C, B), (0, 3, 1, 2)).reshape(N, B * B, C)
    return f.astype(jnp.float32)
